# baseline (speedup 1.0000x reference)
import functools

import jax
import jax.numpy as jnp
from jax import lax
from jax.experimental import pallas as pl
from jax.experimental.pallas import tpu as pltpu
from jax.experimental.pallas import tpu_sc as plsc

_TOTAL = 32768
_NB = 16
_L = 16

_info = plsc.get_sparse_core_info()
_NC = _info.num_cores
_NS = _info.num_subcores
_NW = _NC * _NS
_PPW = _TOTAL // _NW

_mesh = plsc.VectorSubcoreMesh(core_axis_name="c", subcore_axis_name="s")


@functools.partial(
    pl.kernel,
    mesh=_mesh,
    out_type=jax.ShapeDtypeStruct((_TOTAL, 3), jnp.float32),
    compiler_params=pltpu.CompilerParams(
        needs_layout_passes=False, use_tc_tiling_on_sc=False
    ),
    scratch_types=[
        pltpu.VMEM((_NB * 16,), jnp.float32),
        pltpu.VMEM((_PPW, 3), jnp.float32),
        pltpu.VMEM((_PPW,), jnp.int32),
        pltpu.VMEM((_PPW, 3), jnp.float32),
    ],
)
def _se3_sc(tr_hbm, pos_hbm, bat_hbm, out_hbm, tr_v, pos_v, bat_v, out_v):
    wid = lax.axis_index("s") * _NC + lax.axis_index("c")
    pbase = wid * _PPW
    pltpu.sync_copy(tr_hbm, tr_v)
    pltpu.sync_copy(pos_hbm.at[pl.ds(pbase, _PPW)], pos_v)
    pltpu.sync_copy(bat_hbm.at[pl.ds(pbase, _PPW)], bat_v)

    iota = lax.iota(jnp.int32, _L)
    c0 = jnp.full((_L,), 0, jnp.int32)
    c1 = jnp.full((_L,), 1, jnp.int32)
    c2 = jnp.full((_L,), 2, jnp.int32)

    def body(k, carry):
        p = k * _L
        n = iota + p
        b = bat_v[pl.ds(p, _L)]
        t = b * 16
        r00 = plsc.load_gather(tr_v, [t])
        r01 = plsc.load_gather(tr_v, [t + 1])
        r02 = plsc.load_gather(tr_v, [t + 2])
        p0 = plsc.load_gather(tr_v, [t + 3])
        r10 = plsc.load_gather(tr_v, [t + 4])
        r11 = plsc.load_gather(tr_v, [t + 5])
        r12 = plsc.load_gather(tr_v, [t + 6])
        p1 = plsc.load_gather(tr_v, [t + 7])
        r20 = plsc.load_gather(tr_v, [t + 8])
        r21 = plsc.load_gather(tr_v, [t + 9])
        r22 = plsc.load_gather(tr_v, [t + 10])
        p2 = plsc.load_gather(tr_v, [t + 11])
        x = plsc.load_gather(pos_v, [n, c0])
        y = plsc.load_gather(pos_v, [n, c1])
        z = plsc.load_gather(pos_v, [n, c2])
        ox = r00 * x + r01 * y + r02 * z + p0
        oy = r10 * x + r11 * y + r12 * z + p1
        oz = r20 * x + r21 * y + r22 * z + p2
        plsc.store_scatter(out_v, [n, c0], ox)
        plsc.store_scatter(out_v, [n, c1], oy)
        plsc.store_scatter(out_v, [n, c2], oz)
        return carry

    lax.fori_loop(0, _PPW // _L, body, 0)
    pltpu.sync_copy(out_v, out_hbm.at[pl.ds(pbase, _PPW)])


def kernel(trans, pos, batch):
    out = _se3_sc(trans.reshape(-1), pos, batch.astype(jnp.int32))
    return out, batch


# speedup vs baseline: 2.9590x; 2.9590x over previous
import functools

import jax
import jax.numpy as jnp
from jax import lax
from jax.experimental import pallas as pl
from jax.experimental.pallas import tpu as pltpu
from jax.experimental.pallas import tpu_sc as plsc

_TOTAL = 32768
_NB = 16
_L = 16

_info = plsc.get_sparse_core_info()
_NC = _info.num_cores
_NS = _info.num_subcores
_NW = _NC * _NS
_PPW = _TOTAL // _NW

_mesh = plsc.VectorSubcoreMesh(core_axis_name="c", subcore_axis_name="s")


@functools.partial(
    pl.kernel,
    mesh=_mesh,
    out_type=jax.ShapeDtypeStruct((3, _TOTAL), jnp.float32),
    compiler_params=pltpu.CompilerParams(
        needs_layout_passes=False, use_tc_tiling_on_sc=False
    ),
    scratch_types=[
        pltpu.VMEM((_NB * 16,), jnp.float32),
        pltpu.VMEM((3, _PPW), jnp.float32),
        pltpu.VMEM((_PPW,), jnp.int32),
        pltpu.VMEM((3, _PPW), jnp.float32),
    ],
)
def _se3_sc(tr_hbm, pos_hbm, bat_hbm, out_hbm, tr_v, pos_v, bat_v, out_v):
    wid = lax.axis_index("s") * _NC + lax.axis_index("c")
    pbase = wid * _PPW
    pltpu.sync_copy(tr_hbm, tr_v)
    pltpu.sync_copy(pos_hbm.at[:, pl.ds(pbase, _PPW)], pos_v)
    pltpu.sync_copy(bat_hbm.at[pl.ds(pbase, _PPW)], bat_v)

    def body(k, carry):
        p = k * _L
        b = bat_v[pl.ds(p, _L)]
        t = b * 16
        r00 = plsc.load_gather(tr_v, [t])
        r01 = plsc.load_gather(tr_v, [t + 1])
        r02 = plsc.load_gather(tr_v, [t + 2])
        p0 = plsc.load_gather(tr_v, [t + 3])
        r10 = plsc.load_gather(tr_v, [t + 4])
        r11 = plsc.load_gather(tr_v, [t + 5])
        r12 = plsc.load_gather(tr_v, [t + 6])
        p1 = plsc.load_gather(tr_v, [t + 7])
        r20 = plsc.load_gather(tr_v, [t + 8])
        r21 = plsc.load_gather(tr_v, [t + 9])
        r22 = plsc.load_gather(tr_v, [t + 10])
        p2 = plsc.load_gather(tr_v, [t + 11])
        x = pos_v[0, pl.ds(p, _L)]
        y = pos_v[1, pl.ds(p, _L)]
        z = pos_v[2, pl.ds(p, _L)]
        out_v[0, pl.ds(p, _L)] = r00 * x + r01 * y + r02 * z + p0
        out_v[1, pl.ds(p, _L)] = r10 * x + r11 * y + r12 * z + p1
        out_v[2, pl.ds(p, _L)] = r20 * x + r21 * y + r22 * z + p2
        return carry

    lax.fori_loop(0, _PPW // _L, body, 0)
    pltpu.sync_copy(out_v, out_hbm.at[:, pl.ds(pbase, _PPW)])


def kernel(trans, pos, batch):
    outT = _se3_sc(trans.reshape(-1), pos.T, batch.astype(jnp.int32))
    return outT.T, batch


# speedup vs baseline: 2.9620x; 1.0010x over previous
import functools

import jax
import jax.numpy as jnp
from jax import lax
from jax.experimental import pallas as pl
from jax.experimental.pallas import tpu as pltpu
from jax.experimental.pallas import tpu_sc as plsc

_TOTAL = 32768
_NB = 16
_L = 16

_info = plsc.get_sparse_core_info()
_NC = _info.num_cores
_NS = _info.num_subcores
_NW = _NC * _NS
_PPW = _TOTAL // _NW

_mesh = plsc.VectorSubcoreMesh(core_axis_name="c", subcore_axis_name="s")


@functools.partial(
    pl.kernel,
    mesh=_mesh,
    out_type=(
        jax.ShapeDtypeStruct((3, _TOTAL), jnp.float32),
        jax.ShapeDtypeStruct((_TOTAL,), jnp.int32),
    ),
    compiler_params=pltpu.CompilerParams(
        needs_layout_passes=False, use_tc_tiling_on_sc=False
    ),
    scratch_types=[
        pltpu.VMEM((_NB * 16,), jnp.float32),
        pltpu.VMEM((3, _PPW), jnp.float32),
        pltpu.VMEM((_PPW,), jnp.int32),
        pltpu.VMEM((3, _PPW), jnp.float32),
    ],
)
def _se3_sc(tr_hbm, pos_hbm, bat_hbm, out_hbm, bat_out_hbm, tr_v, pos_v, bat_v, out_v):
    wid = lax.axis_index("s") * _NC + lax.axis_index("c")
    pbase = wid * _PPW
    pltpu.sync_copy(tr_hbm, tr_v)
    pltpu.sync_copy(pos_hbm.at[:, pl.ds(pbase, _PPW)], pos_v)
    pltpu.sync_copy(bat_hbm.at[pl.ds(pbase, _PPW)], bat_v)

    @plsc.parallel_loop(0, _PPW // _L, unroll=4)
    def body(k):
        p = k * _L
        b = bat_v[pl.ds(p, _L)]
        t = b * 16
        r00 = plsc.load_gather(tr_v, [t])
        r01 = plsc.load_gather(tr_v, [t + 1])
        r02 = plsc.load_gather(tr_v, [t + 2])
        p0 = plsc.load_gather(tr_v, [t + 3])
        r10 = plsc.load_gather(tr_v, [t + 4])
        r11 = plsc.load_gather(tr_v, [t + 5])
        r12 = plsc.load_gather(tr_v, [t + 6])
        p1 = plsc.load_gather(tr_v, [t + 7])
        r20 = plsc.load_gather(tr_v, [t + 8])
        r21 = plsc.load_gather(tr_v, [t + 9])
        r22 = plsc.load_gather(tr_v, [t + 10])
        p2 = plsc.load_gather(tr_v, [t + 11])
        x = pos_v[0, pl.ds(p, _L)]
        y = pos_v[1, pl.ds(p, _L)]
        z = pos_v[2, pl.ds(p, _L)]
        out_v[0, pl.ds(p, _L)] = r00 * x + r01 * y + r02 * z + p0
        out_v[1, pl.ds(p, _L)] = r10 * x + r11 * y + r12 * z + p1
        out_v[2, pl.ds(p, _L)] = r20 * x + r21 * y + r22 * z + p2

    pltpu.sync_copy(out_v, out_hbm.at[:, pl.ds(pbase, _PPW)])
    pltpu.sync_copy(bat_v, bat_out_hbm.at[pl.ds(pbase, _PPW)])


def kernel(trans, pos, batch):
    outT, new_batch = _se3_sc(trans.reshape(-1), pos.T, batch.astype(jnp.int32))
    return outT.T, new_batch
